# Initial kernel scaffold; baseline (speedup 1.0000x reference)
import jax
import jax.numpy as jnp
from jax.experimental import pallas as pl


def kernel(x, table):
    raise NotImplementedError("write your pallas kernel here")


# speedup vs baseline: 4.4198x; 4.4198x over previous
import functools

import jax
import jax.numpy as jnp
from jax import lax
from jax.experimental import pallas as pl
from jax.experimental.pallas import tpu as pltpu
from jax.experimental.pallas import tpu_sc as plsc

MAX_IDX = 255
D = 32
NC = 2
NS = 16
NW = NC * NS
L = 16
SCH = 512
NBUF = 2


def _sc_body(batch_per_w, T, x_hbm, table_hbm, out_hbm, table_v, idx_v, rows0,
             rows1, ssem0, ssem1):
  wid = lax.axis_index("s") * NC + lax.axis_index("c")
  rows = (rows0, rows1)
  ssems = (ssem0, ssem1)
  rows_per_chunk = SCH // 128

  pltpu.sync_copy(table_hbm, table_v)
  pltpu.sync_copy(x_hbm.at[wid], idx_v)

  @pl.loop(0, T, step=NBUF)
  def _chunks(t):
    for b in range(NBUF):
      c = t + b
      row_b = c >> 1
      soff = (c & 1) * SCH

      @pl.when(t >= NBUF)
      def _():
        pltpu.make_async_copy(rows[b], out_hbm.at[0, :, pl.ds(0, SCH)],
                              ssems[b]).wait()

      @pl.loop(0, SCH // L)
      def _groups(g):
        r = c * rows_per_chunk + (g >> 3)
        off = (g & 7) * L
        idx16 = idx_v[r, pl.ds(off, L)]
        flat = idx16 * D
        for d in range(D):
          val = plsc.load_gather(table_v, [flat + d])
          rows[b][d, pl.ds(g * L, L)] = val

      pltpu.async_copy(rows[b],
                       out_hbm.at[wid * batch_per_w + row_b, :,
                                  pl.ds(soff, SCH)],
                       ssems[b])

  for b in range(NBUF):
    pltpu.make_async_copy(rows[b], out_hbm.at[0, :, pl.ds(0, SCH)],
                          ssems[b]).wait()


def kernel(x, table):
  orig_shape = x.shape
  if x.ndim == 1:
    x = x[None, :]
  batch, seq = x.shape
  assert seq == 2 * SCH and batch % NW == 0
  per_w = batch * seq // NW
  n_rows = per_w // 128
  T = per_w // SCH
  batch_per_w = batch // NW

  x_r = jnp.asarray(x, jnp.int32).reshape(NW, n_rows, 128)
  table_flat = jnp.asarray(table, jnp.float32).reshape(-1)

  mesh = plsc.VectorSubcoreMesh(core_axis_name="c", subcore_axis_name="s")
  k = pl.kernel(
      functools.partial(_sc_body, batch_per_w, T),
      out_type=jax.ShapeDtypeStruct((batch, D, seq), jnp.float32),
      mesh=mesh,
      compiler_params=pltpu.CompilerParams(needs_layout_passes=False),
      scratch_types=[
          pltpu.VMEM(((MAX_IDX + 1) * D,), jnp.float32),
          pltpu.VMEM((n_rows, 128), jnp.int32),
          pltpu.VMEM((D, SCH), jnp.float32),
          pltpu.VMEM((D, SCH), jnp.float32),
          pltpu.SemaphoreType.DMA,
          pltpu.SemaphoreType.DMA,
      ],
  )
  out_t = k(x_r, table_flat)
  return jnp.transpose(out_t, (0, 2, 1)).reshape(orig_shape + (D,))
